# baseline (speedup 1.0000x reference)
import functools

import jax
import jax.numpy as jnp
from jax.experimental import pallas as pl
from jax.experimental.pallas import tpu as pltpu
from jax.experimental.pallas import tpu_sc as plsc

_N = 10000
_D = 256
_M = 512
_K = 3
_BLK = 1000

_NC = 2
_NS = 16
_CHUNK = 64
_CPT = 160
_SLAB = 40
_EPAD = _CPT * _CHUNK * _NS
_HALF = _D // _NC
_AGGROWS = 10240
_NBUF = 4

_sc_mesh = plsc.VectorSubcoreMesh(core_axis_name="c", subcore_axis_name="s",
                                  num_cores=_NC, num_subcores=_NS)


@functools.partial(
    pl.kernel,
    out_type=jax.ShapeDtypeStruct((_NC, _AGGROWS, _HALF), jnp.float32),
    mesh=_sc_mesh,
    scratch_types=[
        pltpu.VMEM((_SLAB, _CHUNK), jnp.int32),
        pltpu.VMEM((_SLAB, _CHUNK), jnp.int32),
        pltpu.VMEM((_CHUNK, _HALF), jnp.float32),
        pltpu.VMEM((_CHUNK, _HALF), jnp.float32),
        pltpu.VMEM((_CHUNK, _HALF), jnp.float32),
        pltpu.VMEM((_CHUNK, _HALF), jnp.float32),
        pltpu.VMEM_SHARED((_AGGROWS, _HALF), jnp.float32),
        pltpu.SemaphoreType.DMA,
        pltpu.SemaphoreType.DMA,
        pltpu.SemaphoreType.DMA,
        pltpu.SemaphoreType.DMA,
    ],
)
def _agg_sc(table_ref, src_ref, dst_ref, zeros_ref, out_ref,
            src_v, dst_v, rb0, rb1, rb2, rb3, agg_sh, g0, g1, g2, g3):
    rows_bufs = [rb0, rb1, rb2, rb3]
    gsems = [g0, g1, g2, g3]
    c = jax.lax.axis_index("c")
    s = jax.lax.axis_index("s")
    pltpu.sync_copy(zeros_ref, rows_bufs[0])
    for i in range(_AGGROWS // _NS // _CHUNK):
        pltpu.sync_copy(rows_bufs[0],
                        agg_sh.at[pl.ds(s * (_AGGROWS // _NS) + i * _CHUNK,
                                        _CHUNK)])
    plsc.subcore_barrier()
    tbl = table_ref.at[c]

    for phase in range(_CPT // _SLAB):
        pltpu.sync_copy(src_ref.at[pl.ds(s * _CPT + phase * _SLAB, _SLAB)],
                        src_v)
        pltpu.sync_copy(dst_ref.at[pl.ds(s * _CPT + phase * _SLAB, _SLAB)],
                        dst_v)
        for bb in range(_NBUF):
            pltpu.async_copy(tbl.at[src_v.at[bb]], rows_bufs[bb], gsems[bb])

        def group(g, carry):
            for bb in range(_NBUF):
                j = g * _NBUF + bb
                pltpu.make_async_copy(tbl.at[src_v.at[j]], rows_bufs[bb],
                                      gsems[bb]).wait()
                pltpu.sync_copy(rows_bufs[bb], agg_sh.at[dst_v.at[j]],
                                add=True)

                @pl.when(j + _NBUF < _SLAB)
                def _():
                    pltpu.async_copy(tbl.at[src_v.at[j + _NBUF]],
                                     rows_bufs[bb], gsems[bb])

            return carry

        jax.lax.fori_loop(0, _SLAB // _NBUF, group, 0)
    plsc.subcore_barrier()
    for i in range(_AGGROWS // _NS // _CHUNK):
        r0 = s * (_AGGROWS // _NS) + i * _CHUNK
        pltpu.sync_copy(agg_sh.at[pl.ds(r0, _CHUNK)], rows_bufs[0])
        pltpu.sync_copy(rows_bufs[0], out_ref.at[c].at[pl.ds(r0, _CHUNK)])


def _hop_body(aggh_ref, eprev_ref, wa_ref, ws_ref, b_ref, cbt_ref,
              cbsq_ref, idx_ref):
    mean = jnp.concatenate([aggh_ref[0], aggh_ref[1]], axis=1)
    eprev = jnp.concatenate([eprev_ref[0], eprev_ref[1]], axis=1)
    lin = (jnp.dot(mean, wa_ref[...],
                   preferred_element_type=jnp.float32)
           + jnp.dot(eprev, ws_ref[...],
                     preferred_element_type=jnp.float32)
           + b_ref[...])
    h = jnp.maximum(lin, 0.0)
    hh = jnp.sum(h * h, axis=1, keepdims=True)
    hc = jnp.dot(h, cbt_ref[...], preferred_element_type=jnp.float32)
    d2 = hh - 2.0 * hc + cbsq_ref[...]
    dmin = jnp.min(d2, axis=1, keepdims=True)
    col = jax.lax.broadcasted_iota(jnp.int32, d2.shape, 1)
    idx = jnp.min(jnp.where(d2 == dmin, col, _M), axis=1)
    idx_ref[...] = idx[:, None]


def _hop(aggh, eprevh, wa, ws, bk, cbt, cbsq):
    grid = _N // _BLK
    idx_p = pl.pallas_call(
        _hop_body,
        grid=(grid,),
        in_specs=[
            pl.BlockSpec((_NC, _BLK, _HALF), lambda i: (0, i, 0)),
            pl.BlockSpec((_NC, _BLK, _HALF), lambda i: (0, i, 0)),
            pl.BlockSpec((_D, _D), lambda i: (0, 0)),
            pl.BlockSpec((_D, _D), lambda i: (0, 0)),
            pl.BlockSpec((1, _D), lambda i: (0, 0)),
            pl.BlockSpec((_D, _M), lambda i: (0, 0)),
            pl.BlockSpec((1, _M), lambda i: (0, 0)),
        ],
        out_specs=pl.BlockSpec((_BLK, 1), lambda i: (i, 0)),
        out_shape=jax.ShapeDtypeStruct((_N, 1), jnp.int32),
    )(aggh, eprevh, wa, ws, bk[None, :], cbt, cbsq)
    return idx_p[:, 0]


def kernel(X, edge_index, Wa, Ws, b, codebooks):
    src = edge_index[0]
    dst = edge_index[1]
    epad = _EPAD - src.shape[0]
    srcm = jnp.concatenate([src, jnp.zeros((epad,), jnp.int32)]).reshape(-1, _CHUNK)
    dstm = jnp.concatenate([dst, jnp.full((epad,), _N, jnp.int32)]).reshape(-1, _CHUNK)
    zeros = jnp.zeros((_CHUNK, _HALF), jnp.float32)
    ones = jnp.ones((src.shape[0], 1), dtype=jnp.float32)
    deg = jax.ops.segment_sum(ones, dst, num_segments=_N)
    degc = jnp.maximum(deg, 1.0)
    eprevh = X.reshape(_N, _NC, _HALF).transpose(1, 0, 2)
    idx_list = []
    idx = None
    for k in range(_K):
        aggh = _agg_sc(eprevh, srcm, dstm, zeros)
        meanh = aggh[:, :_N, :] / degc[None, :, :]
        cb = codebooks[k]
        cbsq = jnp.sum(cb * cb, axis=1)[None, :]
        idx = _hop(meanh, eprevh, Wa[k], Ws[k], b[k], cb.T, cbsq)
        idx_list.append(idx)
        if k + 1 < _K:
            cbh = cb.reshape(_M, _NC, _HALF).transpose(1, 0, 2)
            eprevh = jnp.take(cbh, idx, axis=1)
    e_out = jnp.take(codebooks[_K - 1], idx, axis=0)
    return e_out, jnp.stack(idx_list, axis=0)


# speedup vs baseline: 1.0430x; 1.0430x over previous
import functools

import jax
import jax.numpy as jnp
from jax.experimental import pallas as pl
from jax.experimental.pallas import tpu as pltpu
from jax.experimental.pallas import tpu_sc as plsc

_N = 10000
_D = 256
_M = 512
_K = 3
_BLK = 1000

_NC = 2
_NS = 16
_CHUNK = 64
_CPT = 160
_SLAB = 40
_EPAD = _CPT * _CHUNK * _NS
_HALF = _D // _NC
_AGGROWS = 10240
_NBUF = 4

_sc_mesh = plsc.VectorSubcoreMesh(core_axis_name="c", subcore_axis_name="s",
                                  num_cores=_NC, num_subcores=_NS)


@functools.partial(
    pl.kernel,
    out_type=jax.ShapeDtypeStruct((_NC, _AGGROWS, _HALF), jnp.float32),
    mesh=_sc_mesh,
    scratch_types=[
        pltpu.VMEM((_SLAB, _CHUNK), jnp.int32),
        pltpu.VMEM((_SLAB, _CHUNK), jnp.int32),
        pltpu.VMEM((_CHUNK, _HALF), jnp.float32),
        pltpu.VMEM((_CHUNK, _HALF), jnp.float32),
        pltpu.VMEM((_CHUNK, _HALF), jnp.float32),
        pltpu.VMEM((_CHUNK, _HALF), jnp.float32),
        pltpu.VMEM_SHARED((_AGGROWS, _HALF), jnp.float32),
        pltpu.SemaphoreType.DMA,
        pltpu.SemaphoreType.DMA,
        pltpu.SemaphoreType.DMA,
        pltpu.SemaphoreType.DMA,
    ],
)
def _agg_sc(table_ref, src_ref, dst_ref, zeros_ref, out_ref,
            src_v, dst_v, rb0, rb1, rb2, rb3, agg_sh, g0, g1, g2, g3):
    rows_bufs = [rb0, rb1, rb2, rb3]
    gsems = [g0, g1, g2, g3]
    c = jax.lax.axis_index("c")
    s = jax.lax.axis_index("s")
    pltpu.sync_copy(zeros_ref, rows_bufs[0])
    for i in range(_AGGROWS // _NS // _CHUNK):
        pltpu.sync_copy(rows_bufs[0],
                        agg_sh.at[pl.ds(s * (_AGGROWS // _NS) + i * _CHUNK,
                                        _CHUNK)])
    plsc.subcore_barrier()
    tbl = table_ref.at[c]

    for phase in range(_CPT // _SLAB):
        pltpu.sync_copy(src_ref.at[pl.ds(s * _CPT + phase * _SLAB, _SLAB)],
                        src_v)
        pltpu.sync_copy(dst_ref.at[pl.ds(s * _CPT + phase * _SLAB, _SLAB)],
                        dst_v)
        for bb in range(_NBUF):
            pltpu.async_copy(tbl.at[src_v.at[bb]], rows_bufs[bb], gsems[bb])

        def group(g, carry):
            for bb in range(_NBUF):
                j = g * _NBUF + bb
                pltpu.make_async_copy(tbl.at[src_v.at[j]], rows_bufs[bb],
                                      gsems[bb]).wait()
                pltpu.sync_copy(rows_bufs[bb], agg_sh.at[dst_v.at[j]],
                                add=True)

                @pl.when(j + _NBUF < _SLAB)
                def _():
                    pltpu.async_copy(tbl.at[src_v.at[j + _NBUF]],
                                     rows_bufs[bb], gsems[bb])

            return carry

        jax.lax.fori_loop(0, _SLAB // _NBUF, group, 0)
    plsc.subcore_barrier()
    for i in range(_AGGROWS // _NS // _CHUNK):
        r0 = s * (_AGGROWS // _NS) + i * _CHUNK
        pltpu.sync_copy(agg_sh.at[pl.ds(r0, _CHUNK)], rows_bufs[0])
        pltpu.sync_copy(rows_bufs[0], out_ref.at[c].at[pl.ds(r0, _CHUNK)])


def _hop_body(aggh_ref, eprev_ref, wa_ref, ws_ref, b_ref, cbt_ref,
              cbsq_ref, cb_ref, idx_ref, qout_ref):
    mean = jnp.concatenate([aggh_ref[0], aggh_ref[1]], axis=1)
    eprev = jnp.concatenate([eprev_ref[0], eprev_ref[1]], axis=1)
    lin = (jnp.dot(mean, wa_ref[...],
                   preferred_element_type=jnp.float32)
           + jnp.dot(eprev, ws_ref[...],
                     preferred_element_type=jnp.float32)
           + b_ref[...])
    h = jnp.maximum(lin, 0.0)
    hh = jnp.sum(h * h, axis=1, keepdims=True)
    hc = jnp.dot(h, cbt_ref[...], preferred_element_type=jnp.float32)
    d2 = hh - 2.0 * hc + cbsq_ref[...]
    dmin = jnp.min(d2, axis=1, keepdims=True)
    col = jax.lax.broadcasted_iota(jnp.int32, d2.shape, 1)
    idx = jnp.min(jnp.where(d2 == dmin, col, _M), axis=1)
    idx_ref[...] = idx[:, None]
    onehot = (col == idx[:, None]).astype(jnp.float32)
    quant = jnp.dot(onehot, cb_ref[...], preferred_element_type=jnp.float32,
                    precision=jax.lax.Precision.HIGHEST)
    qout_ref[0] = quant[:, :_HALF]
    qout_ref[1] = quant[:, _HALF:]


def _hop(aggh, eprevh, wa, ws, bk, cbt, cbsq, cb):
    grid = _N // _BLK
    idx_p, quanth = pl.pallas_call(
        _hop_body,
        grid=(grid,),
        in_specs=[
            pl.BlockSpec((_NC, _BLK, _HALF), lambda i: (0, i, 0)),
            pl.BlockSpec((_NC, _BLK, _HALF), lambda i: (0, i, 0)),
            pl.BlockSpec((_D, _D), lambda i: (0, 0)),
            pl.BlockSpec((_D, _D), lambda i: (0, 0)),
            pl.BlockSpec((1, _D), lambda i: (0, 0)),
            pl.BlockSpec((_D, _M), lambda i: (0, 0)),
            pl.BlockSpec((1, _M), lambda i: (0, 0)),
            pl.BlockSpec((_M, _D), lambda i: (0, 0)),
        ],
        out_specs=[
            pl.BlockSpec((_BLK, 1), lambda i: (i, 0)),
            pl.BlockSpec((_NC, _BLK, _HALF), lambda i: (0, i, 0)),
        ],
        out_shape=[
            jax.ShapeDtypeStruct((_N, 1), jnp.int32),
            jax.ShapeDtypeStruct((_NC, _N, _HALF), jnp.float32),
        ],
    )(aggh, eprevh, wa, ws, bk[None, :], cbt, cbsq, cb)
    return idx_p[:, 0], quanth


def kernel(X, edge_index, Wa, Ws, b, codebooks):
    src = edge_index[0]
    dst = edge_index[1]
    epad = _EPAD - src.shape[0]
    srcm = jnp.concatenate([src, jnp.zeros((epad,), jnp.int32)]).reshape(-1, _CHUNK)
    dstm = jnp.concatenate([dst, jnp.full((epad,), _N, jnp.int32)]).reshape(-1, _CHUNK)
    zeros = jnp.zeros((_CHUNK, _HALF), jnp.float32)
    ones = jnp.ones((src.shape[0], 1), dtype=jnp.float32)
    deg = jax.ops.segment_sum(ones, dst, num_segments=_N)
    degc = jnp.maximum(deg, 1.0)
    eprevh = X.reshape(_N, _NC, _HALF).transpose(1, 0, 2)
    idx_list = []
    idx = None
    for k in range(_K):
        aggh = _agg_sc(eprevh, srcm, dstm, zeros)
        meanh = aggh[:, :_N, :] / degc[None, :, :]
        cb = codebooks[k]
        cbsq = jnp.sum(cb * cb, axis=1)[None, :]
        idx, quanth = _hop(meanh, eprevh, Wa[k], Ws[k], b[k], cb.T, cbsq, cb)
        idx_list.append(idx)
        eprevh = quanth
    e_out = eprevh.transpose(1, 0, 2).reshape(_N, _D)
    return e_out, jnp.stack(idx_list, axis=0)
